# Initial kernel scaffold; baseline (speedup 1.0000x reference)
import jax
import jax.numpy as jnp
from jax.experimental import pallas as pl


def kernel(node_feats, edge_feats, edge_index, W_Q, W_K, W_V, W_pe, W_Oh, b_Oh, W_Oe, b_Oe, ln1h_w, ln1h_b, ln1e_w, ln1e_b, W_ffh1, b_ffh1, W_ffh2, b_ffh2, W_ffe1, b_ffe1, W_ffe2, b_ffe2, ln2h_w, ln2h_b, ln2e_w, ln2e_b):
    raise NotImplementedError("write your pallas kernel here")


# speedup vs baseline: 9.9997x; 9.9997x over previous
import functools

import jax
import jax.numpy as jnp
from jax.experimental import pallas as pl

N = 10000
E = 320000
D = 128
H = 8
DH = 16
HID = 128

NODE_BLK = 2000
EDGE_BLK = 2560


def _ln_rows(x, w, b):
    m = x.mean(-1, keepdims=True)
    v = x.var(-1, keepdims=True)
    return (x - m) / jnp.sqrt(v + 1e-5) * w + b


def _qkv_body(x_ref, wq_ref, wk_ref, wv_ref, q_ref, k_ref, v_ref):
    x = x_ref[...]
    q_ref[...] = jnp.dot(x, wq_ref[...], preferred_element_type=jnp.float32)
    k_ref[...] = jnp.dot(x, wk_ref[...], preferred_element_type=jnp.float32)
    v_ref[...] = jnp.dot(x, wv_ref[...], preferred_element_type=jnp.float32)


def _qkv_proj(node_feats, W_Q, W_K, W_V):
    grid = (N // NODE_BLK,)
    blk = pl.BlockSpec((NODE_BLK, D), lambda i: (i, 0))
    wspec = pl.BlockSpec((D, HID), lambda i: (0, 0))
    out = pl.pallas_call(
        _qkv_body,
        grid=grid,
        in_specs=[blk, wspec, wspec, wspec],
        out_specs=[blk, blk, blk],
        out_shape=[jax.ShapeDtypeStruct((N, HID), jnp.float32)] * 3,
    )(node_feats, W_Q, W_K, W_V)
    return out


def _edge_body(ef_in_ref, p_ref, wpe_ref, woe_ref, boe_ref, ln1w_ref, ln1b_ref,
               wf1_ref, bf1_ref, wf2_ref, bf2_ref, ln2w_ref, ln2b_ref,
               seg_ref, ef_out_ref, s_out_ref):
    x = ef_in_ref[...]
    pe = jnp.dot(x, wpe_ref[...], preferred_element_type=jnp.float32)
    score = p_ref[...] * pe * 0.25
    ssum = jnp.dot(score, seg_ref[...], preferred_element_type=jnp.float32)
    s_out_ref[...] = jnp.exp(jnp.clip(ssum, -5.0, 5.0))
    y = jnp.dot(score, woe_ref[...], preferred_element_type=jnp.float32) + boe_ref[...]
    y = _ln_rows(x + y, ln1w_ref[...], ln1b_ref[...])
    y2 = jnp.maximum(jnp.dot(y, wf1_ref[...], preferred_element_type=jnp.float32)
                     + bf1_ref[...], 0.0)
    y2 = jnp.dot(y2, wf2_ref[...], preferred_element_type=jnp.float32) + bf2_ref[...]
    ef_out_ref[...] = _ln_rows(y + y2, ln2w_ref[...], ln2b_ref[...])


def _edge_pipeline(edge_feats, P, W_pe, W_Oe, b_Oe, ln1e_w, ln1e_b,
                   W_ffe1, b_ffe1, W_ffe2, b_ffe2, ln2e_w, ln2e_b):
    grid = (E // EDGE_BLK,)
    blk = pl.BlockSpec((EDGE_BLK, D), lambda i: (i, 0))
    c = lambda *shape: pl.BlockSpec(shape, lambda i: (0,) * len(shape))
    seg = (jnp.arange(D)[:, None] // DH == jnp.arange(H)[None, :]).astype(jnp.float32)
    ef, s = pl.pallas_call(
        _edge_body,
        grid=grid,
        in_specs=[blk, blk, c(D, HID), c(HID, HID), c(HID,), c(HID,), c(HID,),
                  c(HID, 2 * HID), c(2 * HID,), c(2 * HID, HID), c(HID,),
                  c(HID,), c(HID,), c(D, H)],
        out_specs=[blk, pl.BlockSpec((EDGE_BLK, H), lambda i: (i, 0))],
        out_shape=[jax.ShapeDtypeStruct((E, HID), jnp.float32),
                   jax.ShapeDtypeStruct((E, H), jnp.float32)],
    )(edge_feats, P, W_pe, W_Oe, b_Oe, ln1e_w, ln1e_b,
      W_ffe1, b_ffe1, W_ffe2, b_ffe2, ln2e_w, ln2e_b, seg)
    return ef, s


def _node_body(nf_in_ref, wv_ref, z_ref, seg_ref, woh_ref, boh_ref,
               ln1w_ref, ln1b_ref, wf1_ref, bf1_ref, wf2_ref, bf2_ref,
               ln2w_ref, ln2b_ref, nf_out_ref):
    x = nf_in_ref[...]
    zrep = jnp.dot(z_ref[...], seg_ref[...], preferred_element_type=jnp.float32)
    h = wv_ref[...] / (zrep + 1e-6)
    y = jnp.dot(h, woh_ref[...], preferred_element_type=jnp.float32) + boh_ref[...]
    y = _ln_rows(x + y, ln1w_ref[...], ln1b_ref[...])
    y2 = jnp.maximum(jnp.dot(y, wf1_ref[...], preferred_element_type=jnp.float32)
                     + bf1_ref[...], 0.0)
    y2 = jnp.dot(y2, wf2_ref[...], preferred_element_type=jnp.float32) + bf2_ref[...]
    nf_out_ref[...] = _ln_rows(y + y2, ln2w_ref[...], ln2b_ref[...])


def _node_pipeline(node_feats, wV, z, W_Oh, b_Oh, ln1h_w, ln1h_b,
                   W_ffh1, b_ffh1, W_ffh2, b_ffh2, ln2h_w, ln2h_b):
    grid = (N // NODE_BLK,)
    blk = pl.BlockSpec((NODE_BLK, D), lambda i: (i, 0))
    c = lambda *shape: pl.BlockSpec(shape, lambda i: (0,) * len(shape))
    seg = (jnp.arange(H)[:, None] == jnp.arange(D)[None, :] // DH).astype(jnp.float32)
    nf = pl.pallas_call(
        _node_body,
        grid=grid,
        in_specs=[blk, blk, pl.BlockSpec((NODE_BLK, H), lambda i: (i, 0)),
                  c(H, D), c(HID, HID), c(HID,), c(HID,), c(HID,),
                  c(HID, 2 * HID), c(2 * HID,), c(2 * HID, HID), c(HID,),
                  c(HID,), c(HID,)],
        out_specs=blk,
        out_shape=jax.ShapeDtypeStruct((N, HID), jnp.float32),
    )(node_feats, wV, z, seg, W_Oh, b_Oh, ln1h_w, ln1h_b,
      W_ffh1, b_ffh1, W_ffh2, b_ffh2, ln2h_w, ln2h_b)
    return nf


def kernel(node_feats, edge_feats, edge_index, W_Q, W_K, W_V, W_pe, W_Oh, b_Oh,
           W_Oe, b_Oe, ln1h_w, ln1h_b, ln1e_w, ln1e_b, W_ffh1, b_ffh1, W_ffh2,
           b_ffh2, W_ffe1, b_ffe1, W_ffe2, b_ffe2, ln2h_w, ln2h_b, ln2e_w,
           ln2e_b):
    src = edge_index[0]
    dst = edge_index[1]
    Qh, Kh, Vh = _qkv_proj(node_feats, W_Q, W_K, W_V)

    P = jnp.take(Kh, src, axis=0) * jnp.take(Qh, dst, axis=0)

    ef, s = _edge_pipeline(edge_feats, P, W_pe, W_Oe, b_Oe, ln1e_w, ln1e_b,
                           W_ffe1, b_ffe1, W_ffe2, b_ffe2, ln2e_w, ln2e_b)

    sV = (jnp.take(Vh, src, axis=0).reshape(E, H, DH) * s[:, :, None]).reshape(E, HID)
    wV = jax.ops.segment_sum(sV, dst, num_segments=N)
    z = jax.ops.segment_sum(s, dst, num_segments=N)

    nf = _node_pipeline(node_feats, wV, z, W_Oh, b_Oh, ln1h_w, ln1h_b,
                        W_ffh1, b_ffh1, W_ffh2, b_ffh2, ln2h_w, ln2h_b)
    return (nf, ef)


# speedup vs baseline: 16.5808x; 1.6581x over previous
import functools

import jax
import jax.numpy as jnp
from jax import lax
from jax.experimental import pallas as pl
from jax.experimental.pallas import tpu as pltpu
from jax.experimental.pallas import tpu_sc as plsc

N = 10000
E = 320000
D = 128
H = 8
DH = 16
HID = 128

NODE_BLK = 2048
EDGE_BLK = 2560

NC = 2
NS = 16
NW = NC * NS
E_PER_W = E // NW
SC_CHUNK = 80
N_CHUNKS = E_PER_W // SC_CHUNK


def _sc_mesh():
    return plsc.VectorSubcoreMesh(core_axis_name="c", subcore_axis_name="s",
                                  num_cores=NC, num_subcores=NS)


def _gather_prod_body(kh_hbm, qh_hbm, src_hbm, dst_hbm, p_hbm,
                      idx_s, idx_d, krows, qrows, pbuf, sem0, sem1):
    cid = lax.axis_index("c")
    sid = lax.axis_index("s")
    w = sid * NC + cid

    def chunk(k, _):
        base = w * E_PER_W + k * SC_CHUNK
        pltpu.sync_copy(src_hbm.at[pl.ds(base, SC_CHUNK)], idx_s)
        pltpu.sync_copy(dst_hbm.at[pl.ds(base, SC_CHUNK)], idx_d)
        ck = pltpu.async_copy(kh_hbm.at[idx_s], krows, sem0)
        cq = pltpu.async_copy(qh_hbm.at[idx_d], qrows, sem1)
        ck.wait()
        cq.wait()

        def edge(i, _):
            for h in range(H):
                sl = pl.ds(h * DH, DH)
                pbuf[i, sl] = krows[i, sl] * qrows[i, sl]
            return 0

        lax.fori_loop(0, SC_CHUNK, edge, 0)
        pltpu.sync_copy(pbuf, p_hbm.at[pl.ds(base, SC_CHUNK)])
        return 0

    lax.fori_loop(0, N_CHUNKS, chunk, 0)


def _gather_prod(Kh, Qh, src, dst):
    f = pl.kernel(
        _gather_prod_body,
        out_type=jax.ShapeDtypeStruct((E, HID), jnp.float32),
        mesh=_sc_mesh(),
        scratch_types=[
            pltpu.VMEM((SC_CHUNK,), jnp.int32),
            pltpu.VMEM((SC_CHUNK,), jnp.int32),
            pltpu.VMEM((SC_CHUNK, HID), jnp.float32),
            pltpu.VMEM((SC_CHUNK, HID), jnp.float32),
            pltpu.VMEM((SC_CHUNK, HID), jnp.float32),
            pltpu.SemaphoreType.DMA,
            pltpu.SemaphoreType.DMA,
        ],
    )
    return f(Kh, Qh, src, dst)


def _ln_rows(x, w, b):
    m = x.mean(-1, keepdims=True)
    v = x.var(-1, keepdims=True)
    return (x - m) / jnp.sqrt(v + 1e-5) * w + b


def _qkv_body(x_ref, wq_ref, wk_ref, wv_ref, q_ref, k_ref, v_ref):
    x = x_ref[...]
    q_ref[...] = jnp.dot(x, wq_ref[...], preferred_element_type=jnp.float32)
    k_ref[...] = jnp.dot(x, wk_ref[...], preferred_element_type=jnp.float32)
    v_ref[...] = jnp.dot(x, wv_ref[...], preferred_element_type=jnp.float32)


def _qkv_proj(node_feats, W_Q, W_K, W_V):
    grid = (pl.cdiv(N, NODE_BLK),)
    blk = pl.BlockSpec((NODE_BLK, D), lambda i: (i, 0))
    wspec = pl.BlockSpec((D, HID), lambda i: (0, 0))
    out = pl.pallas_call(
        _qkv_body,
        grid=grid,
        in_specs=[blk, wspec, wspec, wspec],
        out_specs=[blk, blk, blk],
        out_shape=[jax.ShapeDtypeStruct((N, HID), jnp.float32)] * 3,
    )(node_feats, W_Q, W_K, W_V)
    return out


NP = 10240
NPH = NP // NC
ACC_W = 256
NPH_T = 5248
ZERO_ROWS = NPH_T // NS
OUT_ROWS = NPH // NS
E_PER_TEC = E // NS
N_CHUNKS_SC = E_PER_TEC // SC_CHUNK
IDX_GRPS = SC_CHUNK // 16


def _scatter_body(vh_hbm, s_hbm, src_hbm, dst_hbm, zeros_hbm, acc_hbm,
                  idx_s, idx_d, vrows, sbuf, val, shared, sem0):
    cid = lax.axis_index("c")
    sid = lax.axis_index("s")
    lo = cid * NPH
    pltpu.sync_copy(zeros_hbm.at[pl.ds(sid * ZERO_ROWS, ZERO_ROWS)],
                    shared.at[pl.ds(sid * ZERO_ROWS, ZERO_ROWS)])
    plsc.subcore_barrier()

    def chunk(k, _):
        base = sid * E_PER_TEC + k * SC_CHUNK
        pltpu.sync_copy(src_hbm.at[pl.ds(base, SC_CHUNK)], idx_s)
        pltpu.sync_copy(dst_hbm.at[pl.ds(base, SC_CHUNK)], idx_d)
        pltpu.sync_copy(s_hbm.at[pl.ds(base, SC_CHUNK)], sbuf)
        pltpu.async_copy(vh_hbm.at[idx_s], vrows, sem0).wait()

        for g in range(IDX_GRPS):
            sl = pl.ds(g * 16, 16)
            local = idx_d[sl] - lo
            oob = (local < 0) | (local >= NPH)
            idx_d[sl] = jnp.where(oob, NPH, local)

        def edge(i, _):
            srow = sbuf[i, :]
            for h in range(H):
                val[i, pl.ds(h * DH, DH)] = vrows[i, pl.ds(h * DH, DH)] * srow
                val[i, pl.ds(HID + h * DH, DH)] = srow
            return 0

        lax.fori_loop(0, SC_CHUNK, edge, 0)
        pltpu.sync_copy(val, shared.at[idx_d], add=True)
        return 0

    lax.fori_loop(0, N_CHUNKS_SC, chunk, 0)
    plsc.subcore_barrier()
    rows = pl.ds(sid * OUT_ROWS, OUT_ROWS)
    pltpu.sync_copy(shared.at[rows], acc_hbm.at[cid, rows])


def _scatter_segsum(Vh, s16, src, dst):
    zeros = jnp.zeros((NPH_T, ACC_W), jnp.float32)
    f = pl.kernel(
        _scatter_body,
        out_type=jax.ShapeDtypeStruct((NC, NPH, ACC_W), jnp.float32),
        mesh=_sc_mesh(),
        compiler_params=pltpu.CompilerParams(use_tc_tiling_on_sc=False),
        scratch_types=[
            pltpu.VMEM((SC_CHUNK,), jnp.int32),
            pltpu.VMEM((SC_CHUNK,), jnp.int32),
            pltpu.VMEM((SC_CHUNK, HID), jnp.float32),
            pltpu.VMEM((SC_CHUNK, DH), jnp.float32),
            pltpu.VMEM((SC_CHUNK, ACC_W), jnp.float32),
            pltpu.VMEM_SHARED((NPH_T, ACC_W), jnp.float32),
            pltpu.SemaphoreType.DMA,
        ],
    )
    return f(Vh, s16, src, dst, zeros)


def _edge_body(ef_in_ref, p_ref, wpe_ref, woe_ref, boe_ref, ln1w_ref, ln1b_ref,
               wf1_ref, bf1_ref, wf2_ref, bf2_ref, ln2w_ref, ln2b_ref,
               seg_ref, ef_out_ref, s_out_ref):
    x = ef_in_ref[...]
    pe = jnp.dot(x, wpe_ref[...], preferred_element_type=jnp.float32)
    score = p_ref[...] * pe * 0.25
    ssum = jnp.dot(score, seg_ref[...], preferred_element_type=jnp.float32)
    s_out_ref[...] = jnp.exp(jnp.clip(ssum, -5.0, 5.0))
    y = jnp.dot(score, woe_ref[...], preferred_element_type=jnp.float32) + boe_ref[...]
    y = _ln_rows(x + y, ln1w_ref[...], ln1b_ref[...])
    y2 = jnp.maximum(jnp.dot(y, wf1_ref[...], preferred_element_type=jnp.float32)
                     + bf1_ref[...], 0.0)
    y2 = jnp.dot(y2, wf2_ref[...], preferred_element_type=jnp.float32) + bf2_ref[...]
    ef_out_ref[...] = _ln_rows(y + y2, ln2w_ref[...], ln2b_ref[...])


def _edge_pipeline(edge_feats, P, W_pe, W_Oe, b_Oe, ln1e_w, ln1e_b,
                   W_ffe1, b_ffe1, W_ffe2, b_ffe2, ln2e_w, ln2e_b):
    grid = (E // EDGE_BLK,)
    blk = pl.BlockSpec((EDGE_BLK, D), lambda i: (i, 0))
    c = lambda *shape: pl.BlockSpec(shape, lambda i: (0,) * len(shape))
    seg = (jnp.arange(D)[:, None] // DH == jnp.arange(DH)[None, :] % H).astype(jnp.float32)
    ef, s = pl.pallas_call(
        _edge_body,
        grid=grid,
        in_specs=[blk, blk, c(D, HID), c(HID, HID), c(HID,), c(HID,), c(HID,),
                  c(HID, 2 * HID), c(2 * HID,), c(2 * HID, HID), c(HID,),
                  c(HID,), c(HID,), c(D, DH)],
        out_specs=[blk, pl.BlockSpec((EDGE_BLK, DH), lambda i: (i, 0))],
        out_shape=[jax.ShapeDtypeStruct((E, HID), jnp.float32),
                   jax.ShapeDtypeStruct((E, DH), jnp.float32)],
    )(edge_feats, P, W_pe, W_Oe, b_Oe, ln1e_w, ln1e_b,
      W_ffe1, b_ffe1, W_ffe2, b_ffe2, ln2e_w, ln2e_b, seg)
    return ef, s


def _node_body(nf_in_ref, acc_ref, woh_ref, boh_ref,
               ln1w_ref, ln1b_ref, wf1_ref, bf1_ref, wf2_ref, bf2_ref,
               ln2w_ref, ln2b_ref, nf_out_ref):
    x = nf_in_ref[...]
    wv = acc_ref[:, 0:HID]
    z = acc_ref[:, HID:ACC_W]
    h = wv / (z + 1e-6)
    y = jnp.dot(h, woh_ref[...], preferred_element_type=jnp.float32) + boh_ref[...]
    y = _ln_rows(x + y, ln1w_ref[...], ln1b_ref[...])
    y2 = jnp.maximum(jnp.dot(y, wf1_ref[...], preferred_element_type=jnp.float32)
                     + bf1_ref[...], 0.0)
    y2 = jnp.dot(y2, wf2_ref[...], preferred_element_type=jnp.float32) + bf2_ref[...]
    nf_out_ref[...] = _ln_rows(y + y2, ln2w_ref[...], ln2b_ref[...])


def _node_pipeline(node_feats, acc, W_Oh, b_Oh, ln1h_w, ln1h_b,
                   W_ffh1, b_ffh1, W_ffh2, b_ffh2, ln2h_w, ln2h_b):
    grid = (NP // NODE_BLK,)
    blk = pl.BlockSpec((NODE_BLK, D), lambda i: (i, 0))
    c = lambda *shape: pl.BlockSpec(shape, lambda i: (0,) * len(shape))
    nf = pl.pallas_call(
        _node_body,
        grid=grid,
        in_specs=[blk, pl.BlockSpec((NODE_BLK, ACC_W), lambda i: (i, 0)),
                  c(HID, HID), c(HID,), c(HID,), c(HID,),
                  c(HID, 2 * HID), c(2 * HID,), c(2 * HID, HID), c(HID,),
                  c(HID,), c(HID,)],
        out_specs=blk,
        out_shape=jax.ShapeDtypeStruct((N, HID), jnp.float32),
    )(node_feats, acc.reshape(NP, ACC_W), W_Oh, b_Oh, ln1h_w, ln1h_b,
      W_ffh1, b_ffh1, W_ffh2, b_ffh2, ln2h_w, ln2h_b)
    return nf


def kernel(node_feats, edge_feats, edge_index, W_Q, W_K, W_V, W_pe, W_Oh, b_Oh,
           W_Oe, b_Oe, ln1h_w, ln1h_b, ln1e_w, ln1e_b, W_ffh1, b_ffh1, W_ffh2,
           b_ffh2, W_ffe1, b_ffe1, W_ffe2, b_ffe2, ln2h_w, ln2h_b, ln2e_w,
           ln2e_b):
    src = edge_index[0]
    dst = edge_index[1]
    W_Vt = W_V.reshape(D, H, DH).transpose(0, 2, 1).reshape(D, HID)
    W_Oht = W_Oh.reshape(H, DH, HID).transpose(1, 0, 2).reshape(HID, HID)
    Qh, Kh, Vh = _qkv_proj(node_feats, W_Q, W_K, W_Vt)

    P = _gather_prod(Kh, Qh, src, dst)

    ef, s = _edge_pipeline(edge_feats, P, W_pe, W_Oe, b_Oe, ln1e_w, ln1e_b,
                           W_ffe1, b_ffe1, W_ffe2, b_ffe2, ln2e_w, ln2e_b)

    acc = _scatter_segsum(Vh, s, src, dst)

    nf = _node_pipeline(node_feats, acc, W_Oht, b_Oh, ln1h_w, ln1h_b,
                        W_ffh1, b_ffh1, W_ffh2, b_ffh2, ln2h_w, ln2h_b)
    return (nf, ef)


# speedup vs baseline: 24.0378x; 1.4497x over previous
import functools

import jax
import jax.numpy as jnp
from jax import lax
from jax.experimental import pallas as pl
from jax.experimental.pallas import tpu as pltpu
from jax.experimental.pallas import tpu_sc as plsc

N = 10000
E = 320000
D = 128
H = 8
DH = 16
HID = 128

NODE_BLK = 2048
EDGE_BLK = 2560

NC = 2
NS = 16
NW = NC * NS
E_PER_W = E // NW
SC_CHUNK = 80
N_CHUNKS = E_PER_W // SC_CHUNK


def _sc_mesh():
    return plsc.VectorSubcoreMesh(core_axis_name="c", subcore_axis_name="s",
                                  num_cores=NC, num_subcores=NS)


def _gather_prod_body(kh_hbm, qh_hbm, vh_hbm, src_hbm, dst_hbm, p_hbm, vsrc_hbm,
                      idx_s, idx_d, krows, qrows, vrows, pbuf, sem0, sem1, sem2):
    cid = lax.axis_index("c")
    sid = lax.axis_index("s")
    w = sid * NC + cid

    def chunk(k, _):
        base = w * E_PER_W + k * SC_CHUNK
        pltpu.sync_copy(src_hbm.at[pl.ds(base, SC_CHUNK)], idx_s)
        pltpu.sync_copy(dst_hbm.at[pl.ds(base, SC_CHUNK)], idx_d)
        ck = pltpu.async_copy(kh_hbm.at[idx_s], krows, sem0)
        cq = pltpu.async_copy(qh_hbm.at[idx_d], qrows, sem1)
        cv = pltpu.async_copy(vh_hbm.at[idx_s], vrows, sem2)
        ck.wait()
        cq.wait()

        def edge(i, _):
            for h in range(H):
                sl = pl.ds(h * DH, DH)
                pbuf[i, sl] = krows[i, sl] * qrows[i, sl]
            return 0

        lax.fori_loop(0, SC_CHUNK, edge, 0)
        pltpu.sync_copy(pbuf, p_hbm.at[pl.ds(base, SC_CHUNK)])
        cv.wait()
        pltpu.sync_copy(vrows, vsrc_hbm.at[pl.ds(base, SC_CHUNK)])
        return 0

    lax.fori_loop(0, N_CHUNKS, chunk, 0)


def _gather_prod(Kh, Qh, Vh, src, dst):
    f = pl.kernel(
        _gather_prod_body,
        out_type=[jax.ShapeDtypeStruct((E, HID), jnp.float32),
                  jax.ShapeDtypeStruct((E, HID), jnp.float32)],
        mesh=_sc_mesh(),
        scratch_types=[
            pltpu.VMEM((SC_CHUNK,), jnp.int32),
            pltpu.VMEM((SC_CHUNK,), jnp.int32),
            pltpu.VMEM((SC_CHUNK, HID), jnp.float32),
            pltpu.VMEM((SC_CHUNK, HID), jnp.float32),
            pltpu.VMEM((SC_CHUNK, HID), jnp.float32),
            pltpu.VMEM((SC_CHUNK, HID), jnp.float32),
            pltpu.SemaphoreType.DMA,
            pltpu.SemaphoreType.DMA,
            pltpu.SemaphoreType.DMA,
        ],
    )
    return f(Kh, Qh, Vh, src, dst)


def _ln_rows(x, w, b):
    m = x.mean(-1, keepdims=True)
    v = x.var(-1, keepdims=True)
    return (x - m) / jnp.sqrt(v + 1e-5) * w + b


def _qkv_body(x_ref, wq_ref, wk_ref, wv_ref, q_ref, k_ref, v_ref):
    x = x_ref[...]
    q_ref[...] = jnp.dot(x, wq_ref[...], preferred_element_type=jnp.float32)
    k_ref[...] = jnp.dot(x, wk_ref[...], preferred_element_type=jnp.float32)
    v_ref[...] = jnp.dot(x, wv_ref[...], preferred_element_type=jnp.float32)


def _qkv_proj(node_feats, W_Q, W_K, W_V):
    grid = (pl.cdiv(N, NODE_BLK),)
    blk = pl.BlockSpec((NODE_BLK, D), lambda i: (i, 0))
    wspec = pl.BlockSpec((D, HID), lambda i: (0, 0))
    out = pl.pallas_call(
        _qkv_body,
        grid=grid,
        in_specs=[blk, wspec, wspec, wspec],
        out_specs=[blk, blk, blk],
        out_shape=[jax.ShapeDtypeStruct((N, HID), jnp.float32)] * 3,
    )(node_feats, W_Q, W_K, W_V)
    return out


NP = 10240
NPS = NP // NS
E_PER_CORE = E // NC
E_PC_S = E_PER_CORE // NS
N_CHUNKS_SC = E_PC_S // SC_CHUNK


def _scatter_body(wv_hbm, s_hbm, dst_hbm, zwv_hbm, zz_hbm, awv_hbm, az_hbm,
                  idx_d, wvbuf, sbuf, tbl_wv, tbl_z):
    cid = lax.axis_index("c")
    sid = lax.axis_index("s")
    rows = pl.ds(sid * NPS, NPS)
    pltpu.sync_copy(zwv_hbm.at[rows], tbl_wv.at[rows])
    pltpu.sync_copy(zz_hbm.at[rows], tbl_z.at[rows])
    plsc.subcore_barrier()

    def chunk(k, _):
        base = cid * E_PER_CORE + sid * E_PC_S + k * SC_CHUNK
        pltpu.sync_copy(dst_hbm.at[pl.ds(base, SC_CHUNK)], idx_d)
        pltpu.sync_copy(wv_hbm.at[pl.ds(base, SC_CHUNK)], wvbuf)
        pltpu.sync_copy(s_hbm.at[pl.ds(base, SC_CHUNK)], sbuf)
        pltpu.sync_copy(wvbuf, tbl_wv.at[idx_d], add=True)
        pltpu.sync_copy(sbuf, tbl_z.at[idx_d], add=True)
        return 0

    lax.fori_loop(0, N_CHUNKS_SC, chunk, 0)
    plsc.subcore_barrier()
    pltpu.sync_copy(tbl_wv.at[rows], awv_hbm.at[cid, rows])
    pltpu.sync_copy(tbl_z.at[rows], az_hbm.at[cid, rows])


def _scatter_segsum(wv, s16, dst):
    zwv = jnp.zeros((NP, HID), jnp.float32)
    zz = jnp.zeros((NP, DH), jnp.float32)
    f = pl.kernel(
        _scatter_body,
        out_type=[jax.ShapeDtypeStruct((NC, NP, HID), jnp.float32),
                  jax.ShapeDtypeStruct((NC, NP, DH), jnp.float32)],
        mesh=_sc_mesh(),
        compiler_params=pltpu.CompilerParams(use_tc_tiling_on_sc=False),
        scratch_types=[
            pltpu.VMEM((SC_CHUNK,), jnp.int32),
            pltpu.VMEM((SC_CHUNK, HID), jnp.float32),
            pltpu.VMEM((SC_CHUNK, DH), jnp.float32),
            pltpu.VMEM_SHARED((NP, HID), jnp.float32),
            pltpu.VMEM_SHARED((NP, DH), jnp.float32),
        ],
    )
    return f(wv, s16, dst, zwv, zz)


def _edge_body(ef_in_ref, p_ref, vsrc_ref, wpe_ref, woe_ref, boe_ref,
               ln1w_ref, ln1b_ref, wf1_ref, bf1_ref, wf2_ref, bf2_ref,
               ln2w_ref, ln2b_ref, seg_ref, ef_out_ref, s_out_ref, wv_out_ref):
    x = ef_in_ref[...]
    pe = jnp.dot(x, wpe_ref[...], preferred_element_type=jnp.float32)
    score = p_ref[...] * pe * 0.25
    ssum = jnp.dot(score, seg_ref[...], preferred_element_type=jnp.float32)
    s = jnp.exp(jnp.clip(ssum, -5.0, 5.0))
    s_out_ref[...] = s
    s128 = jnp.concatenate([s] * (HID // DH), axis=1)
    wv_out_ref[...] = vsrc_ref[...] * s128
    y = jnp.dot(score, woe_ref[...], preferred_element_type=jnp.float32) + boe_ref[...]
    y = _ln_rows(x + y, ln1w_ref[...], ln1b_ref[...])
    y2 = jnp.maximum(jnp.dot(y, wf1_ref[...], preferred_element_type=jnp.float32)
                     + bf1_ref[...], 0.0)
    y2 = jnp.dot(y2, wf2_ref[...], preferred_element_type=jnp.float32) + bf2_ref[...]
    ef_out_ref[...] = _ln_rows(y + y2, ln2w_ref[...], ln2b_ref[...])


def _edge_pipeline(edge_feats, P, Vsrc, W_pe, W_Oe, b_Oe, ln1e_w, ln1e_b,
                   W_ffe1, b_ffe1, W_ffe2, b_ffe2, ln2e_w, ln2e_b):
    grid = (E // EDGE_BLK,)
    blk = pl.BlockSpec((EDGE_BLK, D), lambda i: (i, 0))
    c = lambda *shape: pl.BlockSpec(shape, lambda i: (0,) * len(shape))
    seg = (jnp.arange(D)[:, None] // DH == jnp.arange(DH)[None, :] % H).astype(jnp.float32)
    ef, s, wv = pl.pallas_call(
        _edge_body,
        grid=grid,
        in_specs=[blk, blk, blk, c(D, HID), c(HID, HID), c(HID,), c(HID,),
                  c(HID,), c(HID, 2 * HID), c(2 * HID,), c(2 * HID, HID),
                  c(HID,), c(HID,), c(HID,), c(D, DH)],
        out_specs=[blk, pl.BlockSpec((EDGE_BLK, DH), lambda i: (i, 0)), blk],
        out_shape=[jax.ShapeDtypeStruct((E, HID), jnp.float32),
                   jax.ShapeDtypeStruct((E, DH), jnp.float32),
                   jax.ShapeDtypeStruct((E, HID), jnp.float32)],
    )(edge_feats, P, Vsrc, W_pe, W_Oe, b_Oe, ln1e_w, ln1e_b,
      W_ffe1, b_ffe1, W_ffe2, b_ffe2, ln2e_w, ln2e_b, seg)
    return ef, s, wv


def _node_body(nf_in_ref, awv_ref, az_ref, woh_ref, boh_ref,
               ln1w_ref, ln1b_ref, wf1_ref, bf1_ref, wf2_ref, bf2_ref,
               ln2w_ref, ln2b_ref, nf_out_ref):
    x = nf_in_ref[...]
    wv = awv_ref[0] + awv_ref[1]
    z16 = az_ref[0] + az_ref[1]
    z = jnp.concatenate([z16] * (HID // DH), axis=1)
    h = wv / (z + 1e-6)
    y = jnp.dot(h, woh_ref[...], preferred_element_type=jnp.float32) + boh_ref[...]
    y = _ln_rows(x + y, ln1w_ref[...], ln1b_ref[...])
    y2 = jnp.maximum(jnp.dot(y, wf1_ref[...], preferred_element_type=jnp.float32)
                     + bf1_ref[...], 0.0)
    y2 = jnp.dot(y2, wf2_ref[...], preferred_element_type=jnp.float32) + bf2_ref[...]
    nf_out_ref[...] = _ln_rows(y + y2, ln2w_ref[...], ln2b_ref[...])


def _node_pipeline(node_feats, awv, az, W_Oh, b_Oh, ln1h_w, ln1h_b,
                   W_ffh1, b_ffh1, W_ffh2, b_ffh2, ln2h_w, ln2h_b):
    grid = (NP // NODE_BLK,)
    blk = pl.BlockSpec((NODE_BLK, D), lambda i: (i, 0))
    c = lambda *shape: pl.BlockSpec(shape, lambda i: (0,) * len(shape))
    nf = pl.pallas_call(
        _node_body,
        grid=grid,
        in_specs=[blk, pl.BlockSpec((NC, NODE_BLK, HID), lambda i: (0, i, 0)),
                  pl.BlockSpec((NC, NODE_BLK, DH), lambda i: (0, i, 0)),
                  c(HID, HID), c(HID,), c(HID,), c(HID,),
                  c(HID, 2 * HID), c(2 * HID,), c(2 * HID, HID), c(HID,),
                  c(HID,), c(HID,)],
        out_specs=blk,
        out_shape=jax.ShapeDtypeStruct((N, HID), jnp.float32),
    )(node_feats, awv, az, W_Oh, b_Oh, ln1h_w, ln1h_b,
      W_ffh1, b_ffh1, W_ffh2, b_ffh2, ln2h_w, ln2h_b)
    return nf


def kernel(node_feats, edge_feats, edge_index, W_Q, W_K, W_V, W_pe, W_Oh, b_Oh,
           W_Oe, b_Oe, ln1h_w, ln1h_b, ln1e_w, ln1e_b, W_ffh1, b_ffh1, W_ffh2,
           b_ffh2, W_ffe1, b_ffe1, W_ffe2, b_ffe2, ln2h_w, ln2h_b, ln2e_w,
           ln2e_b):
    src = edge_index[0]
    dst = edge_index[1]
    W_Vt = W_V.reshape(D, H, DH).transpose(0, 2, 1).reshape(D, HID)
    W_Oht = W_Oh.reshape(H, DH, HID).transpose(1, 0, 2).reshape(HID, HID)
    Qh, Kh, Vh = _qkv_proj(node_feats, W_Q, W_K, W_Vt)

    P, Vsrc = _gather_prod(Kh, Qh, Vh, src, dst)

    ef, s, wv = _edge_pipeline(edge_feats, P, Vsrc, W_pe, W_Oe, b_Oe,
                               ln1e_w, ln1e_b, W_ffe1, b_ffe1, W_ffe2, b_ffe2,
                               ln2e_w, ln2e_b)

    awv, az = _scatter_segsum(wv, s, dst)

    nf = _node_pipeline(node_feats, awv, az, W_Oht, b_Oh, ln1h_w, ln1h_b,
                        W_ffh1, b_ffh1, W_ffh2, b_ffh2, ln2h_w, ln2h_b)
    return (nf, ef)


# speedup vs baseline: 27.3120x; 1.1362x over previous
import functools

import jax
import jax.numpy as jnp
from jax import lax
from jax.experimental import pallas as pl
from jax.experimental.pallas import tpu as pltpu
from jax.experimental.pallas import tpu_sc as plsc

N = 10000
E = 320000
D = 128
H = 8
DH = 16
HID = 128

NODE_BLK = 2048
EDGE_BLK = 2560

NC = 2
NS = 16
NW = NC * NS
E_PER_W = E // NW
SC_CHUNK = 80
N_CHUNKS = E_PER_W // SC_CHUNK


def _sc_mesh():
    return plsc.VectorSubcoreMesh(core_axis_name="c", subcore_axis_name="s",
                                  num_cores=NC, num_subcores=NS)


def _gather_prod_body(kh_hbm, qh_hbm, vh_hbm, src_hbm, dst_hbm, p_hbm, vsrc_hbm,
                      idx_s, idx_d, krows, qrows, vrows, pbuf, sem0, sem1, sem2):
    cid = lax.axis_index("c")
    sid = lax.axis_index("s")
    w = sid * NC + cid

    def chunk(k, _):
        base = w * E_PER_W + k * SC_CHUNK
        pltpu.sync_copy(src_hbm.at[pl.ds(base, SC_CHUNK)], idx_s)
        pltpu.sync_copy(dst_hbm.at[pl.ds(base, SC_CHUNK)], idx_d)
        ck = pltpu.async_copy(kh_hbm.at[idx_s], krows, sem0)
        cq = pltpu.async_copy(qh_hbm.at[idx_d], qrows, sem1)
        cv = pltpu.async_copy(vh_hbm.at[idx_s], vrows, sem2)
        ck.wait()
        cq.wait()

        def edge(i, _):
            for h in range(H):
                sl = pl.ds(h * DH, DH)
                pbuf[i, sl] = krows[i, sl] * qrows[i, sl]
            return 0

        lax.fori_loop(0, SC_CHUNK, edge, 0)
        pltpu.sync_copy(pbuf, p_hbm.at[pl.ds(base, SC_CHUNK)])
        cv.wait()
        pltpu.sync_copy(vrows, vsrc_hbm.at[pl.ds(base, SC_CHUNK)])
        return 0

    lax.fori_loop(0, N_CHUNKS, chunk, 0)


def _gather_prod(Kh, Qh, Vh, src, dst):
    f = pl.kernel(
        _gather_prod_body,
        out_type=[jax.ShapeDtypeStruct((E, HID), jnp.float32),
                  jax.ShapeDtypeStruct((E, HID), jnp.float32)],
        mesh=_sc_mesh(),
        scratch_types=[
            pltpu.VMEM((SC_CHUNK,), jnp.int32),
            pltpu.VMEM((SC_CHUNK,), jnp.int32),
            pltpu.VMEM((SC_CHUNK, HID), jnp.float32),
            pltpu.VMEM((SC_CHUNK, HID), jnp.float32),
            pltpu.VMEM((SC_CHUNK, HID), jnp.float32),
            pltpu.VMEM((SC_CHUNK, HID), jnp.float32),
            pltpu.SemaphoreType.DMA,
            pltpu.SemaphoreType.DMA,
            pltpu.SemaphoreType.DMA,
        ],
    )
    return f(Kh, Qh, Vh, src, dst)


def _ln_rows(x, w, b):
    m = x.mean(-1, keepdims=True)
    v = x.var(-1, keepdims=True)
    return (x - m) / jnp.sqrt(v + 1e-5) * w + b


def _qkv_body(x_ref, wq_ref, wk_ref, wv_ref, q_ref, k_ref, v_ref):
    x = x_ref[...]
    q_ref[...] = jnp.dot(x, wq_ref[...], preferred_element_type=jnp.float32)
    k_ref[...] = jnp.dot(x, wk_ref[...], preferred_element_type=jnp.float32)
    v_ref[...] = jnp.dot(x, wv_ref[...], preferred_element_type=jnp.float32)


def _qkv_proj(node_feats, W_Q, W_K, W_V):
    grid = (pl.cdiv(N, NODE_BLK),)
    blk = pl.BlockSpec((NODE_BLK, D), lambda i: (i, 0))
    wspec = pl.BlockSpec((D, HID), lambda i: (0, 0))
    out = pl.pallas_call(
        _qkv_body,
        grid=grid,
        in_specs=[blk, wspec, wspec, wspec],
        out_specs=[blk, blk, blk],
        out_shape=[jax.ShapeDtypeStruct((N, HID), jnp.float32)] * 3,
    )(node_feats, W_Q, W_K, W_V)
    return out


NP = 10240
NPS = NP // NS
E_PER_CORE = E // NC
E_PC_S = E_PER_CORE // NS
N_CHUNKS_SC = E_PC_S // SC_CHUNK


def _scatter_body(wv_hbm, s_hbm, dst_hbm, zwv_hbm, zz_hbm, awv_hbm, az_hbm,
                  idx_d, wvbuf, sbuf, tbl_wv, tbl_z):
    cid = lax.axis_index("c")
    sid = lax.axis_index("s")
    rows = pl.ds(sid * NPS, NPS)
    pltpu.sync_copy(zwv_hbm.at[rows], tbl_wv.at[rows])
    pltpu.sync_copy(zz_hbm.at[rows], tbl_z.at[rows])
    plsc.subcore_barrier()

    def chunk(k, _):
        base = cid * E_PER_CORE + sid * E_PC_S + k * SC_CHUNK
        pltpu.sync_copy(dst_hbm.at[pl.ds(base, SC_CHUNK)], idx_d)
        pltpu.sync_copy(wv_hbm.at[pl.ds(base, SC_CHUNK)], wvbuf)
        pltpu.sync_copy(s_hbm.at[pl.ds(base, SC_CHUNK)], sbuf)
        pltpu.sync_copy(wvbuf, tbl_wv.at[idx_d], add=True)
        pltpu.sync_copy(sbuf, tbl_z.at[idx_d], add=True)
        return 0

    lax.fori_loop(0, N_CHUNKS_SC, chunk, 0)
    plsc.subcore_barrier()
    pltpu.sync_copy(tbl_wv.at[rows], awv_hbm.at[cid, rows])
    pltpu.sync_copy(tbl_z.at[rows], az_hbm.at[cid, rows])


def _scatter_segsum(wv, s16, dst):
    zwv = jnp.zeros((NP, HID), jnp.float32)
    zz = jnp.zeros((NP, DH), jnp.float32)
    f = pl.kernel(
        _scatter_body,
        out_type=[jax.ShapeDtypeStruct((NC, NP, HID), jnp.float32),
                  jax.ShapeDtypeStruct((NC, NP, DH), jnp.float32)],
        mesh=_sc_mesh(),
        compiler_params=pltpu.CompilerParams(use_tc_tiling_on_sc=False),
        scratch_types=[
            pltpu.VMEM((SC_CHUNK,), jnp.int32),
            pltpu.VMEM((SC_CHUNK, HID), jnp.float32),
            pltpu.VMEM((SC_CHUNK, DH), jnp.float32),
            pltpu.VMEM_SHARED((NP, HID), jnp.float32),
            pltpu.VMEM_SHARED((NP, DH), jnp.float32),
        ],
    )
    return f(wv, s16, dst, zwv, zz)


def _edge_score_body(ef_in_ref, p_ref, vsrc_ref, wpe_ref, seg_ref,
                     score_out_ref, s_out_ref, wv_out_ref):
    x = ef_in_ref[...]
    pe = jnp.dot(x, wpe_ref[...], preferred_element_type=jnp.float32)
    score = p_ref[...] * pe * 0.25
    score_out_ref[...] = score
    ssum = jnp.dot(score, seg_ref[...], preferred_element_type=jnp.float32)
    s = jnp.exp(jnp.clip(ssum, -5.0, 5.0))
    s_out_ref[...] = s
    s128 = jnp.concatenate([s] * (HID // DH), axis=1)
    wv_out_ref[...] = vsrc_ref[...] * s128


def _edge_score(edge_feats, P, Vsrc, W_pe):
    grid = (E // EDGE_BLK,)
    blk = pl.BlockSpec((EDGE_BLK, D), lambda i: (i, 0))
    c = lambda *shape: pl.BlockSpec(shape, lambda i: (0,) * len(shape))
    seg = (jnp.arange(D)[:, None] // DH == jnp.arange(DH)[None, :] % H).astype(jnp.float32)
    score, s, wv = pl.pallas_call(
        _edge_score_body,
        grid=grid,
        in_specs=[blk, blk, blk, c(D, HID), c(D, DH)],
        out_specs=[blk, pl.BlockSpec((EDGE_BLK, DH), lambda i: (i, 0)), blk],
        out_shape=[jax.ShapeDtypeStruct((E, HID), jnp.float32),
                   jax.ShapeDtypeStruct((E, DH), jnp.float32),
                   jax.ShapeDtypeStruct((E, HID), jnp.float32)],
    )(edge_feats, P, Vsrc, W_pe, seg)
    return score, s, wv


def _edge_ffn_body(ef_in_ref, score_ref, woe_ref, boe_ref, ln1w_ref, ln1b_ref,
                   wf1_ref, bf1_ref, wf2_ref, bf2_ref, ln2w_ref, ln2b_ref,
                   ef_out_ref):
    x = ef_in_ref[...]
    score = score_ref[...]
    y = jnp.dot(score, woe_ref[...], preferred_element_type=jnp.float32) + boe_ref[...]
    y = _ln_rows(x + y, ln1w_ref[...], ln1b_ref[...])
    y2 = jnp.maximum(jnp.dot(y, wf1_ref[...], preferred_element_type=jnp.float32)
                     + bf1_ref[...], 0.0)
    y2 = jnp.dot(y2, wf2_ref[...], preferred_element_type=jnp.float32) + bf2_ref[...]
    ef_out_ref[...] = _ln_rows(y + y2, ln2w_ref[...], ln2b_ref[...])


def _edge_ffn(edge_feats, score, W_Oe, b_Oe, ln1e_w, ln1e_b,
              W_ffe1, b_ffe1, W_ffe2, b_ffe2, ln2e_w, ln2e_b):
    grid = (E // EDGE_BLK,)
    blk = pl.BlockSpec((EDGE_BLK, D), lambda i: (i, 0))
    c = lambda *shape: pl.BlockSpec(shape, lambda i: (0,) * len(shape))
    ef = pl.pallas_call(
        _edge_ffn_body,
        grid=grid,
        in_specs=[blk, blk, c(HID, HID), c(HID,), c(HID,), c(HID,),
                  c(HID, 2 * HID), c(2 * HID,), c(2 * HID, HID), c(HID,),
                  c(HID,), c(HID,)],
        out_specs=blk,
        out_shape=jax.ShapeDtypeStruct((E, HID), jnp.float32),
    )(edge_feats, score, W_Oe, b_Oe, ln1e_w, ln1e_b,
      W_ffe1, b_ffe1, W_ffe2, b_ffe2, ln2e_w, ln2e_b)
    return ef


def _node_body(nf_in_ref, awv_ref, az_ref, woh_ref, boh_ref,
               ln1w_ref, ln1b_ref, wf1_ref, bf1_ref, wf2_ref, bf2_ref,
               ln2w_ref, ln2b_ref, nf_out_ref):
    x = nf_in_ref[...]
    wv = awv_ref[0] + awv_ref[1]
    z16 = az_ref[0] + az_ref[1]
    z = jnp.concatenate([z16] * (HID // DH), axis=1)
    h = wv / (z + 1e-6)
    y = jnp.dot(h, woh_ref[...], preferred_element_type=jnp.float32) + boh_ref[...]
    y = _ln_rows(x + y, ln1w_ref[...], ln1b_ref[...])
    y2 = jnp.maximum(jnp.dot(y, wf1_ref[...], preferred_element_type=jnp.float32)
                     + bf1_ref[...], 0.0)
    y2 = jnp.dot(y2, wf2_ref[...], preferred_element_type=jnp.float32) + bf2_ref[...]
    nf_out_ref[...] = _ln_rows(y + y2, ln2w_ref[...], ln2b_ref[...])


def _node_pipeline(node_feats, awv, az, W_Oh, b_Oh, ln1h_w, ln1h_b,
                   W_ffh1, b_ffh1, W_ffh2, b_ffh2, ln2h_w, ln2h_b):
    grid = (NP // NODE_BLK,)
    blk = pl.BlockSpec((NODE_BLK, D), lambda i: (i, 0))
    c = lambda *shape: pl.BlockSpec(shape, lambda i: (0,) * len(shape))
    nf = pl.pallas_call(
        _node_body,
        grid=grid,
        in_specs=[blk, pl.BlockSpec((NC, NODE_BLK, HID), lambda i: (0, i, 0)),
                  pl.BlockSpec((NC, NODE_BLK, DH), lambda i: (0, i, 0)),
                  c(HID, HID), c(HID,), c(HID,), c(HID,),
                  c(HID, 2 * HID), c(2 * HID,), c(2 * HID, HID), c(HID,),
                  c(HID,), c(HID,)],
        out_specs=blk,
        out_shape=jax.ShapeDtypeStruct((N, HID), jnp.float32),
    )(node_feats, awv, az, W_Oh, b_Oh, ln1h_w, ln1h_b,
      W_ffh1, b_ffh1, W_ffh2, b_ffh2, ln2h_w, ln2h_b)
    return nf


def kernel(node_feats, edge_feats, edge_index, W_Q, W_K, W_V, W_pe, W_Oh, b_Oh,
           W_Oe, b_Oe, ln1h_w, ln1h_b, ln1e_w, ln1e_b, W_ffh1, b_ffh1, W_ffh2,
           b_ffh2, W_ffe1, b_ffe1, W_ffe2, b_ffe2, ln2h_w, ln2h_b, ln2e_w,
           ln2e_b):
    src = edge_index[0]
    dst = edge_index[1]
    W_Vt = W_V.reshape(D, H, DH).transpose(0, 2, 1).reshape(D, HID)
    W_Oht = W_Oh.reshape(H, DH, HID).transpose(1, 0, 2).reshape(HID, HID)
    Qh, Kh, Vh = _qkv_proj(node_feats, W_Q, W_K, W_Vt)

    P, Vsrc = _gather_prod(Kh, Qh, Vh, src, dst)

    score, s, wv = _edge_score(edge_feats, P, Vsrc, W_pe)

    awv, az = _scatter_segsum(wv, s, dst)

    ef = _edge_ffn(edge_feats, score, W_Oe, b_Oe, ln1e_w, ln1e_b,
                   W_ffe1, b_ffe1, W_ffe2, b_ffe2, ln2e_w, ln2e_b)

    nf = _node_pipeline(node_feats, awv, az, W_Oht, b_Oh, ln1h_w, ln1h_b,
                        W_ffh1, b_ffh1, W_ffh2, b_ffh2, ln2h_w, ln2h_b)
    return (nf, ef)
